# Initial kernel scaffold; baseline (speedup 1.0000x reference)
import functools

import jax
import jax.numpy as jnp
from jax import lax
from jax.experimental import pallas as pl
from jax.experimental.pallas import tpu as pltpu
from jax.experimental.pallas import tpu_sc as plsc

N_NODES = 10000
N_EDGES = 320000
N_WORKERS = 32
E_PER_W = N_EDGES // N_WORKERS
CHUNK = 80
N_CHUNKS = E_PER_W // CHUNK
ROWS_PER_TILE = N_NODES // 16

_MESH = dict(core_axis_name="c", subcore_axis_name="s")


def _sc_deg_body(dst_hbm, out_hbm, dstv, hist):
    c = lax.axis_index("c")
    s = lax.axis_index("s")
    wid = s * 2 + c
    pltpu.sync_copy(dst_hbm.at[wid], dstv)

    def zero(i, _):
        hist[pl.ds(i * 16, 16)] = jnp.zeros((16,), jnp.int32)
        return 0

    lax.fori_loop(0, N_NODES // 16, zero, 0)

    ones = jnp.ones((16,), jnp.int32)

    def step(i, _):
        idx = dstv[pl.ds(i * 16, 16)]
        plsc.addupdate_scatter(hist, [idx], ones)
        return 0

    lax.fori_loop(0, E_PER_W // 16, step, 0)
    pltpu.sync_copy(hist, out_hbm.at[wid])


def _sc_deg(dst32):
    k = pl.kernel(
        _sc_deg_body,
        out_type=jax.ShapeDtypeStruct((N_WORKERS, N_NODES), jnp.int32),
        scratch_types=[
            pltpu.VMEM((E_PER_W,), jnp.int32),
            pltpu.VMEM((N_NODES,), jnp.int32),
        ],
        mesh=plsc.VectorSubcoreMesh(**_MESH),
    )
    return k(dst32)


def _sc_agg_body(g_hbm, src_hbm, dst_hbm, out_hbm, srcv, dstv, buf, acc, sem):
    c = lax.axis_index("c")
    s = lax.axis_index("s")
    wid = s * 2 + c
    pltpu.sync_copy(src_hbm.at[wid], srcv)
    pltpu.sync_copy(dst_hbm.at[wid], dstv)

    def zbuf(r, _):
        for k in range(8):
            buf[r, pl.ds(k * 16, 16)] = jnp.zeros((16,), jnp.float32)
        return 0

    lax.fori_loop(0, CHUNK, zbuf, 0)
    base = s * ROWS_PER_TILE
    for t in range(7):
        pltpu.sync_copy(buf, acc.at[pl.ds(base + t * CHUNK, CHUNK)])
    pltpu.sync_copy(buf.at[pl.ds(0, 65)], acc.at[pl.ds(base + 560, 65)])
    plsc.subcore_barrier()

    def chunk(j, _):
        pltpu.async_copy(g_hbm.at[srcv.at[j]], buf, sem).wait()
        pltpu.sync_copy(buf, acc.at[dstv.at[j]], add=True)
        return 0

    lax.fori_loop(0, N_CHUNKS, chunk, 0)
    plsc.subcore_barrier()
    pltpu.sync_copy(acc.at[pl.ds(base, ROWS_PER_TILE)], out_hbm.at[c, s])


def _sc_agg(g, src32, dst32):
    k = pl.kernel(
        _sc_agg_body,
        out_type=jax.ShapeDtypeStruct((2, 16, ROWS_PER_TILE, 128), jnp.float32),
        scratch_types=[
            pltpu.VMEM((N_CHUNKS, CHUNK), jnp.int32),
            pltpu.VMEM((N_CHUNKS, CHUNK), jnp.int32),
            pltpu.VMEM((CHUNK, 128), jnp.float32),
            pltpu.VMEM_SHARED((N_NODES, 128), jnp.float32),
            pltpu.SemaphoreType.DMA,
        ],
        mesh=plsc.VectorSubcoreMesh(**_MESH),
    )
    return k(g, src32, dst32)


def _tc_dinv_body(h_ref, o_ref):
    deg = jnp.sum(h_ref[...], axis=0).astype(jnp.float32) + 1.0
    o_ref[...] = lax.rsqrt(deg)


def _tc_dinv(hists):
    return pl.pallas_call(
        _tc_dinv_body,
        out_shape=jax.ShapeDtypeStruct((N_NODES,), jnp.float32),
    )(hists)


_BLK = 1000


def _tc_scale_body(d_ref, x_ref, o_ref):
    o_ref[...] = d_ref[...] * x_ref[...]


def _tc_scale(dinv_col, x):
    grid = (N_NODES // _BLK,)
    return pl.pallas_call(
        _tc_scale_body,
        grid=grid,
        in_specs=[
            pl.BlockSpec((_BLK, 1), lambda i: (i, 0)),
            pl.BlockSpec((_BLK, 128), lambda i: (i, 0)),
        ],
        out_specs=pl.BlockSpec((_BLK, 128), lambda i: (i, 0)),
        out_shape=jax.ShapeDtypeStruct((N_NODES, 128), jnp.float32),
    )(dinv_col, x)


def _tc_mid_body(p0, p1, g1, d, w1, bb1, w2, o_ref):
    a = d[...] * (p0[...] + p1[...] + g1[...])
    h = jnp.dot(a, w1[...], preferred_element_type=jnp.float32) + bb1[...]
    h = jnp.maximum(h, 0.0)
    t = jnp.dot(h, w2[...], preferred_element_type=jnp.float32)
    o_ref[...] = d[...] * t


def _tc_mid(p0, p1, g1, dinv_col, W1, b1, W2):
    grid = (N_NODES // _BLK,)
    row = lambda i: (i, 0)
    full = lambda i: (0, 0)
    return pl.pallas_call(
        _tc_mid_body,
        grid=grid,
        in_specs=[
            pl.BlockSpec((_BLK, 128), row),
            pl.BlockSpec((_BLK, 128), row),
            pl.BlockSpec((_BLK, 128), row),
            pl.BlockSpec((_BLK, 1), row),
            pl.BlockSpec((128, 256), full),
            pl.BlockSpec((1, 256), full),
            pl.BlockSpec((256, 128), full),
        ],
        out_specs=pl.BlockSpec((_BLK, 128), row),
        out_shape=jax.ShapeDtypeStruct((N_NODES, 128), jnp.float32),
    )(p0, p1, g1, dinv_col, W1, b1.reshape(1, 256), W2)


def _tc_post_body(q0, q1, g2, d, bb2, o_ref):
    a = d[...] * (q0[...] + q1[...] + g2[...]) + bb2[...]
    m = jnp.max(a, axis=-1, keepdims=True)
    e = jnp.exp(a - m)
    o_ref[...] = e / jnp.sum(e, axis=-1, keepdims=True)


def _tc_post(q0, q1, g2, dinv_col, b2):
    grid = (N_NODES // _BLK,)
    row = lambda i: (i, 0)
    full = lambda i: (0, 0)
    return pl.pallas_call(
        _tc_post_body,
        grid=grid,
        in_specs=[
            pl.BlockSpec((_BLK, 128), row),
            pl.BlockSpec((_BLK, 128), row),
            pl.BlockSpec((_BLK, 128), row),
            pl.BlockSpec((_BLK, 1), row),
            pl.BlockSpec((1, 128), full),
        ],
        out_specs=pl.BlockSpec((_BLK, 128), row),
        out_shape=jax.ShapeDtypeStruct((N_NODES, 128), jnp.float32),
    )(q0, q1, g2, dinv_col, b2.reshape(1, 128))


def kernel(x, edge_index, W1, b1, W2, b2):
    src = edge_index[0].astype(jnp.int32)
    dst = edge_index[1].astype(jnp.int32)
    src_r = src.reshape(N_WORKERS, N_CHUNKS, CHUNK)
    dst_r = dst.reshape(N_WORKERS, N_CHUNKS, CHUNK)
    dst_flat = dst.reshape(N_WORKERS, E_PER_W)

    hists = _sc_deg(dst_flat)
    dinv = _tc_dinv(hists)
    dinv_col = dinv.reshape(N_NODES, 1)

    g1 = _tc_scale(dinv_col, x)
    p = _sc_agg(g1, src_r, dst_r).reshape(2, N_NODES, 128)
    g2 = _tc_mid(p[0], p[1], g1, dinv_col, W1, b1, W2)
    q = _sc_agg(g2, src_r, dst_r).reshape(2, N_NODES, 128)
    out = _tc_post(q[0], q[1], g2, dinv_col, b2)
    return out


# speedup vs baseline: 21.0941x; 21.0941x over previous
import functools

import jax
import jax.numpy as jnp
from jax import lax
from jax.experimental import pallas as pl
from jax.experimental.pallas import tpu as pltpu
from jax.experimental.pallas import tpu_sc as plsc

N_NODES = 10000
N_EDGES = 320000
N_WORKERS = 32
E_PER_W = N_EDGES // N_WORKERS
CHUNK = 80
N_CHUNKS = E_PER_W // CHUNK
ROWS_PER_TILE = N_NODES // 16

_MESH = dict(core_axis_name="c", subcore_axis_name="s")
_SC_PARAMS = pltpu.CompilerParams(needs_layout_passes=False)


def _sc_deg_body(dst_hbm, out_hbm, dstv, hist):
    c = lax.axis_index("c")
    s = lax.axis_index("s")
    wid = s * 2 + c
    pltpu.sync_copy(dst_hbm.at[wid], dstv)

    def zero(i, _):
        hist[pl.ds(i * 16, 16)] = jnp.zeros((16,), jnp.int32)
        return 0

    lax.fori_loop(0, N_NODES // 16, zero, 0)

    ones = jnp.ones((16,), jnp.int32)

    def step(i, _):
        idx = dstv[pl.ds(i * 16, 16)]
        plsc.addupdate_scatter(hist, [idx], ones)
        return 0

    lax.fori_loop(0, E_PER_W // 16, step, 0)
    pltpu.sync_copy(hist, out_hbm.at[wid])


def _sc_deg(dst32):
    k = pl.kernel(
        _sc_deg_body,
        out_type=jax.ShapeDtypeStruct((N_WORKERS, N_NODES), jnp.int32),
        scratch_types=[
            pltpu.VMEM((E_PER_W,), jnp.int32),
            pltpu.VMEM((N_NODES,), jnp.int32),
        ],
        mesh=plsc.VectorSubcoreMesh(**_MESH),
        compiler_params=_SC_PARAMS,
    )
    return k(dst32)


def _sc_agg_body(g_hbm, src_hbm, dst_hbm, out_hbm, srcv, dstv, buf, acc, sem):
    c = lax.axis_index("c")
    s = lax.axis_index("s")
    wid = s * 2 + c
    pltpu.sync_copy(src_hbm.at[wid], srcv)
    pltpu.sync_copy(dst_hbm.at[wid], dstv)

    def zbuf(r, _):
        for k in range(8):
            buf[r, pl.ds(k * 16, 16)] = jnp.zeros((16,), jnp.float32)
        return 0

    lax.fori_loop(0, CHUNK, zbuf, 0)
    base = s * ROWS_PER_TILE
    for t in range(7):
        pltpu.sync_copy(buf, acc.at[pl.ds(base + t * CHUNK, CHUNK)])
    pltpu.sync_copy(buf.at[pl.ds(0, 65)], acc.at[pl.ds(base + 560, 65)])
    plsc.subcore_barrier()

    def chunk(j, _):
        pltpu.async_copy(g_hbm.at[srcv.at[j]], buf, sem).wait()
        pltpu.sync_copy(buf, acc.at[dstv.at[j]], add=True)
        return 0

    lax.fori_loop(0, N_CHUNKS, chunk, 0)
    plsc.subcore_barrier()
    pltpu.sync_copy(acc.at[pl.ds(base, ROWS_PER_TILE)], out_hbm.at[c, s])


def _sc_agg(g, src32, dst32):
    k = pl.kernel(
        _sc_agg_body,
        out_type=jax.ShapeDtypeStruct((2, 16, ROWS_PER_TILE, 128), jnp.float32),
        scratch_types=[
            pltpu.VMEM((N_CHUNKS, CHUNK), jnp.int32),
            pltpu.VMEM((N_CHUNKS, CHUNK), jnp.int32),
            pltpu.VMEM((CHUNK, 128), jnp.float32),
            pltpu.VMEM_SHARED((N_NODES, 128), jnp.float32),
            pltpu.SemaphoreType.DMA,
        ],
        mesh=plsc.VectorSubcoreMesh(**_MESH),
        compiler_params=_SC_PARAMS,
    )
    return k(g, src32, dst32)


def _tc_dinv_body(h_ref, o_ref):
    deg = jnp.sum(h_ref[...], axis=0).astype(jnp.float32) + 1.0
    o_ref[...] = lax.rsqrt(deg)


def _tc_dinv(hists):
    return pl.pallas_call(
        _tc_dinv_body,
        out_shape=jax.ShapeDtypeStruct((N_NODES,), jnp.float32),
    )(hists)


_BLK = 1000


def _tc_scale_body(d_ref, x_ref, o_ref):
    o_ref[...] = d_ref[...] * x_ref[...]


def _tc_scale(dinv_col, x):
    grid = (N_NODES // _BLK,)
    return pl.pallas_call(
        _tc_scale_body,
        grid=grid,
        in_specs=[
            pl.BlockSpec((_BLK, 1), lambda i: (i, 0)),
            pl.BlockSpec((_BLK, 128), lambda i: (i, 0)),
        ],
        out_specs=pl.BlockSpec((_BLK, 128), lambda i: (i, 0)),
        out_shape=jax.ShapeDtypeStruct((N_NODES, 128), jnp.float32),
    )(dinv_col, x)


def _tc_mid_body(p0, p1, g1, d, w1, bb1, w2, o_ref):
    a = d[...] * (p0[...] + p1[...] + g1[...])
    h = jnp.dot(a, w1[...], preferred_element_type=jnp.float32) + bb1[...]
    h = jnp.maximum(h, 0.0)
    t = jnp.dot(h, w2[...], preferred_element_type=jnp.float32)
    o_ref[...] = d[...] * t


def _tc_mid(p0, p1, g1, dinv_col, W1, b1, W2):
    grid = (N_NODES // _BLK,)
    row = lambda i: (i, 0)
    full = lambda i: (0, 0)
    return pl.pallas_call(
        _tc_mid_body,
        grid=grid,
        in_specs=[
            pl.BlockSpec((_BLK, 128), row),
            pl.BlockSpec((_BLK, 128), row),
            pl.BlockSpec((_BLK, 128), row),
            pl.BlockSpec((_BLK, 1), row),
            pl.BlockSpec((128, 256), full),
            pl.BlockSpec((1, 256), full),
            pl.BlockSpec((256, 128), full),
        ],
        out_specs=pl.BlockSpec((_BLK, 128), row),
        out_shape=jax.ShapeDtypeStruct((N_NODES, 128), jnp.float32),
    )(p0, p1, g1, dinv_col, W1, b1.reshape(1, 256), W2)


def _tc_post_body(q0, q1, g2, d, bb2, o_ref):
    a = d[...] * (q0[...] + q1[...] + g2[...]) + bb2[...]
    m = jnp.max(a, axis=-1, keepdims=True)
    e = jnp.exp(a - m)
    o_ref[...] = e / jnp.sum(e, axis=-1, keepdims=True)


def _tc_post(q0, q1, g2, dinv_col, b2):
    grid = (N_NODES // _BLK,)
    row = lambda i: (i, 0)
    full = lambda i: (0, 0)
    return pl.pallas_call(
        _tc_post_body,
        grid=grid,
        in_specs=[
            pl.BlockSpec((_BLK, 128), row),
            pl.BlockSpec((_BLK, 128), row),
            pl.BlockSpec((_BLK, 128), row),
            pl.BlockSpec((_BLK, 1), row),
            pl.BlockSpec((1, 128), full),
        ],
        out_specs=pl.BlockSpec((_BLK, 128), row),
        out_shape=jax.ShapeDtypeStruct((N_NODES, 128), jnp.float32),
    )(q0, q1, g2, dinv_col, b2.reshape(1, 128))


def kernel(x, edge_index, W1, b1, W2, b2):
    src = edge_index[0].astype(jnp.int32)
    dst = edge_index[1].astype(jnp.int32)
    src_r = src.reshape(N_WORKERS, N_CHUNKS, CHUNK)
    dst_r = dst.reshape(N_WORKERS, N_CHUNKS, CHUNK)
    dst_flat = dst.reshape(N_WORKERS, E_PER_W)

    hists = _sc_deg(dst_flat)
    dinv = _tc_dinv(hists)
    dinv_col = dinv.reshape(N_NODES, 1)

    g1 = _tc_scale(dinv_col, x)
    p = _sc_agg(g1, src_r, dst_r).reshape(2, N_NODES, 128)
    g2 = _tc_mid(p[0], p[1], g1, dinv_col, W1, b1, W2)
    q = _sc_agg(g2, src_r, dst_r).reshape(2, N_NODES, 128)
    out = _tc_post(q[0], q[1], g2, dinv_col, b2)
    return out


# speedup vs baseline: 31.6725x; 1.5015x over previous
import functools

import jax
import jax.numpy as jnp
from jax import lax
from jax.experimental import pallas as pl
from jax.experimental.pallas import tpu as pltpu
from jax.experimental.pallas import tpu_sc as plsc

N_NODES = 10000
N_EDGES = 320000
N_WORKERS = 32
E_PER_W = N_EDGES // N_WORKERS
CHUNK = 80
N_CHUNKS = E_PER_W // CHUNK
ROWS_PER_TILE = N_NODES // 16

_MESH = dict(core_axis_name="c", subcore_axis_name="s")
_SC_PARAMS = pltpu.CompilerParams(needs_layout_passes=False)


def _sc_deg_body(dst_hbm, out_hbm, dstv, hist):
    c = lax.axis_index("c")
    s = lax.axis_index("s")
    wid = s * 2 + c
    pltpu.sync_copy(dst_hbm.at[wid], dstv)

    def zero(i, _):
        hist[pl.ds(i * 16, 16)] = jnp.zeros((16,), jnp.int32)
        return 0

    lax.fori_loop(0, N_NODES // 16, zero, 0)

    ones = jnp.ones((16,), jnp.int32)

    def step(i, _):
        idx = dstv[pl.ds(i * 16, 16)]
        plsc.addupdate_scatter(hist, [idx], ones)
        return 0

    lax.fori_loop(0, E_PER_W // 16, step, 0)
    pltpu.sync_copy(hist, out_hbm.at[wid])


def _sc_deg(dst32):
    k = pl.kernel(
        _sc_deg_body,
        out_type=jax.ShapeDtypeStruct((N_WORKERS, N_NODES), jnp.int32),
        scratch_types=[
            pltpu.VMEM((E_PER_W,), jnp.int32),
            pltpu.VMEM((N_NODES,), jnp.int32),
        ],
        mesh=plsc.VectorSubcoreMesh(**_MESH),
        compiler_params=_SC_PARAMS,
    )
    return k(dst32)


def _sc_agg_body(g_hbm, src_hbm, dst_hbm, out_hbm, srcv, dst0, dst1,
                 buf0, buf1, acc, sem0, sem1, semd0, semd1):
    c = lax.axis_index("c")
    s = lax.axis_index("s")
    wid = s * 2 + c
    pltpu.sync_copy(src_hbm.at[wid], srcv)

    def zbuf(r, _):
        for k in range(8):
            buf0[r, pl.ds(k * 16, 16)] = jnp.zeros((16,), jnp.float32)
        return 0

    lax.fori_loop(0, CHUNK, zbuf, 0)
    base = s * ROWS_PER_TILE
    for t in range(7):
        pltpu.sync_copy(buf0, acc.at[pl.ds(base + t * CHUNK, CHUNK)])
    pltpu.sync_copy(buf0.at[pl.ds(0, 65)], acc.at[pl.ds(base + 560, 65)])
    plsc.subcore_barrier()

    def g_at(j):
        return g_hbm.at[srcv.at[pl.ds(j * CHUNK, CHUNK)]]

    pltpu.async_copy(g_at(0), buf0, sem0)
    pltpu.async_copy(dst_hbm.at[wid * N_CHUNKS], dst0, semd0)

    def pair(i, _):
        j = 2 * i
        pltpu.async_copy(g_at(j + 1), buf1, sem1)
        pltpu.async_copy(dst_hbm.at[wid * N_CHUNKS + j + 1], dst1, semd1)
        pltpu.make_async_copy(g_at(j), buf0, sem0).wait()
        pltpu.make_async_copy(dst_hbm.at[wid * N_CHUNKS + j], dst0, semd0).wait()
        pltpu.sync_copy(buf0, acc.at[dst0], add=True)
        pltpu.async_copy(g_at(j + 2), buf0, sem0)
        pltpu.async_copy(dst_hbm.at[wid * N_CHUNKS + j + 2], dst0, semd0)
        pltpu.make_async_copy(g_at(j + 1), buf1, sem1).wait()
        pltpu.make_async_copy(dst_hbm.at[wid * N_CHUNKS + j + 1], dst1, semd1).wait()
        pltpu.sync_copy(buf1, acc.at[dst1], add=True)
        return 0

    lax.fori_loop(0, (N_CHUNKS - 1) // 2, pair, 0)
    j = N_CHUNKS - 1
    pltpu.make_async_copy(g_at(j), buf0, sem0).wait()
    pltpu.make_async_copy(dst_hbm.at[wid * N_CHUNKS + j], dst0, semd0).wait()
    pltpu.sync_copy(buf0, acc.at[dst0], add=True)
    plsc.subcore_barrier()
    pltpu.sync_copy(acc.at[pl.ds(base, ROWS_PER_TILE)], out_hbm.at[c, s])


def _sc_agg(g, src32, dst32):
    k = pl.kernel(
        _sc_agg_body,
        out_type=jax.ShapeDtypeStruct((2, 16, ROWS_PER_TILE, 128), jnp.float32),
        scratch_types=[
            pltpu.VMEM((E_PER_W,), jnp.int32),
            pltpu.VMEM((CHUNK,), jnp.int32),
            pltpu.VMEM((CHUNK,), jnp.int32),
            pltpu.VMEM((CHUNK, 128), jnp.float32),
            pltpu.VMEM((CHUNK, 128), jnp.float32),
            pltpu.VMEM_SHARED((N_NODES, 128), jnp.float32),
            pltpu.SemaphoreType.DMA,
            pltpu.SemaphoreType.DMA,
            pltpu.SemaphoreType.DMA,
            pltpu.SemaphoreType.DMA,
        ],
        mesh=plsc.VectorSubcoreMesh(**_MESH),
        compiler_params=_SC_PARAMS,
    )
    return k(g, src32, dst32)


def _tc_dinv_body(h_ref, o_ref):
    deg = jnp.sum(h_ref[...], axis=0).astype(jnp.float32) + 1.0
    o_ref[...] = lax.rsqrt(deg)


def _tc_dinv(hists):
    return pl.pallas_call(
        _tc_dinv_body,
        out_shape=jax.ShapeDtypeStruct((N_NODES,), jnp.float32),
    )(hists)


_BLK = 1000


def _tc_scale_body(d_ref, x_ref, o_ref):
    o_ref[...] = d_ref[...] * x_ref[...]


def _tc_scale(dinv_col, x):
    grid = (N_NODES // _BLK,)
    return pl.pallas_call(
        _tc_scale_body,
        grid=grid,
        in_specs=[
            pl.BlockSpec((_BLK, 1), lambda i: (i, 0)),
            pl.BlockSpec((_BLK, 128), lambda i: (i, 0)),
        ],
        out_specs=pl.BlockSpec((_BLK, 128), lambda i: (i, 0)),
        out_shape=jax.ShapeDtypeStruct((N_NODES, 128), jnp.float32),
    )(dinv_col, x)


def _tc_mid_body(p0, p1, g1, d, w1, bb1, w2, o_ref):
    a = d[...] * (p0[...] + p1[...] + g1[...])
    h = jnp.dot(a, w1[...], preferred_element_type=jnp.float32) + bb1[...]
    h = jnp.maximum(h, 0.0)
    t = jnp.dot(h, w2[...], preferred_element_type=jnp.float32)
    o_ref[...] = d[...] * t


def _tc_mid(p0, p1, g1, dinv_col, W1, b1, W2):
    grid = (N_NODES // _BLK,)
    row = lambda i: (i, 0)
    full = lambda i: (0, 0)
    return pl.pallas_call(
        _tc_mid_body,
        grid=grid,
        in_specs=[
            pl.BlockSpec((_BLK, 128), row),
            pl.BlockSpec((_BLK, 128), row),
            pl.BlockSpec((_BLK, 128), row),
            pl.BlockSpec((_BLK, 1), row),
            pl.BlockSpec((128, 256), full),
            pl.BlockSpec((1, 256), full),
            pl.BlockSpec((256, 128), full),
        ],
        out_specs=pl.BlockSpec((_BLK, 128), row),
        out_shape=jax.ShapeDtypeStruct((N_NODES, 128), jnp.float32),
    )(p0, p1, g1, dinv_col, W1, b1.reshape(1, 256), W2)


def _tc_post_body(q0, q1, g2, d, bb2, o_ref):
    a = d[...] * (q0[...] + q1[...] + g2[...]) + bb2[...]
    m = jnp.max(a, axis=-1, keepdims=True)
    e = jnp.exp(a - m)
    o_ref[...] = e / jnp.sum(e, axis=-1, keepdims=True)


def _tc_post(q0, q1, g2, dinv_col, b2):
    grid = (N_NODES // _BLK,)
    row = lambda i: (i, 0)
    full = lambda i: (0, 0)
    return pl.pallas_call(
        _tc_post_body,
        grid=grid,
        in_specs=[
            pl.BlockSpec((_BLK, 128), row),
            pl.BlockSpec((_BLK, 128), row),
            pl.BlockSpec((_BLK, 128), row),
            pl.BlockSpec((_BLK, 1), row),
            pl.BlockSpec((1, 128), full),
        ],
        out_specs=pl.BlockSpec((_BLK, 128), row),
        out_shape=jax.ShapeDtypeStruct((N_NODES, 128), jnp.float32),
    )(q0, q1, g2, dinv_col, b2.reshape(1, 128))


def kernel(x, edge_index, W1, b1, W2, b2):
    src = edge_index[0].astype(jnp.int32)
    dst = edge_index[1].astype(jnp.int32)
    src_r = src.reshape(N_WORKERS, E_PER_W)
    dst_r = dst.reshape(N_WORKERS * N_CHUNKS, CHUNK)
    dst_flat = dst.reshape(N_WORKERS, E_PER_W)

    hists = _sc_deg(dst_flat)
    dinv = _tc_dinv(hists)
    dinv_col = dinv.reshape(N_NODES, 1)

    g1 = _tc_scale(dinv_col, x)
    p = _sc_agg(g1, src_r, dst_r).reshape(2, N_NODES, 128)
    g2 = _tc_mid(p[0], p[1], g1, dinv_col, W1, b1, W2)
    q = _sc_agg(g2, src_r, dst_r).reshape(2, N_NODES, 128)
    out = _tc_post(q[0], q[1], g2, dinv_col, b2)
    return out
